# Initial kernel scaffold; baseline (speedup 1.0000x reference)
import jax
import jax.numpy as jnp
from jax.experimental import pallas as pl


def kernel(x, edge_index, edge_weights, weight):
    raise NotImplementedError("write your pallas kernel here")


# speedup vs baseline: 141.3051x; 141.3051x over previous
import functools

import jax
import jax.numpy as jnp
from jax import lax
from jax.experimental import pallas as pl
from jax.experimental.pallas import tpu as pltpu
from jax.experimental.pallas import tpu_sc as plsc

N = 100000
E = 6400000
ROWS = E // 128
CHUNK_ROWS = 16
NCHUNKS = ROWS // CHUNK_ROWS
NWORKERS = 32
ITERS = -(-NCHUNKS // NWORKERS)
CB = 2000
NCB = N // CB
OUT_F = 64
KH = 5
PT = 3200
SUB = 400

_mesh = plsc.VectorSubcoreMesh(core_axis_name="c", subcore_axis_name="s",
                               num_cores=2)

_params = pltpu.CompilerParams(needs_layout_passes=False)

_f32 = jnp.float32


@functools.partial(
    pl.kernel,
    out_type=(jax.ShapeDtypeStruct((N,), _f32),
              jax.ShapeDtypeStruct((N,), _f32),
              jax.ShapeDtypeStruct((N,), _f32)),
    mesh=_mesh,
    scratch_types=[
        pltpu.VMEM((N,), _f32),
        pltpu.VMEM((CB,), _f32),
        pltpu.VMEM((CHUNK_ROWS, 128), jnp.int32),
        pltpu.VMEM((CHUNK_ROWS, 128), jnp.int32),
        pltpu.VMEM((CHUNK_ROWS, 128), _f32),
        pltpu.VMEM((CHUNK_ROWS, 128), _f32),
        pltpu.VMEM((3200,), _f32),
        pltpu.VMEM_SHARED((N,), _f32),
        pltpu.SemaphoreType.DMA,
    ],
    compiler_params=_params,
)
def _round(y0_hbm, y1_hbm, src_hbm, dst_hbm, ew_hbm,
           o_y0, o_y1, o_conv,
           x_v, tmp_v, src_v, dst_v, ew_v, msg_v, z_v, acc_sh, sem):
    c = lax.axis_index("c")
    s = lax.axis_index("s")
    wid = s * 2 + c

    pltpu.sync_copy(y0_hbm, x_v)

    def combine_body(cb, carry):
        base = cb * CB
        pltpu.sync_copy(y1_hbm.at[pl.ds(base, CB)], tmp_v)
        for g in range(CB // 16):
            sl_x = pl.ds(base + g * 16, 16)
            sl_t = pl.ds(g * 16, 16)
            x_v[sl_x] = x_v[sl_x] + tmp_v[sl_t]
        return carry

    lax.fori_loop(0, NCB, combine_body, 0)

    @pl.when(wid < 31)
    def _():
        off = wid * PT
        pltpu.sync_copy(x_v.at[pl.ds(off, PT)], o_conv.at[pl.ds(off, PT)])

    @pl.when(wid == 31)
    def _():
        pltpu.sync_copy(x_v.at[pl.ds(31 * PT, N - 31 * PT)],
                        o_conv.at[pl.ds(31 * PT, N - 31 * PT)])

    def z_body(i, carry):
        z_v[pl.ds(i * 16, 16)] = jnp.zeros((16,), _f32)
        return carry

    lax.fori_loop(0, 3200 // 16, z_body, 0)

    @pl.when(s < 15)
    def _():
        pltpu.sync_copy(z_v, acc_sh.at[pl.ds(s * 6400, 3200)])
        pltpu.sync_copy(z_v, acc_sh.at[pl.ds(s * 6400 + 3200, 3200)])

    @pl.when(s == 15)
    def _():
        pltpu.sync_copy(z_v, acc_sh.at[pl.ds(96000, 3200)])
        pltpu.sync_copy(z_v.at[pl.ds(0, 800)], acc_sh.at[pl.ds(99200, 800)])

    plsc.subcore_barrier()

    def chunk_body(i, carry):
        t = i * NWORKERS + wid

        @pl.when(t < NCHUNKS)
        def _():
            row0 = t * CHUNK_ROWS
            pltpu.sync_copy(src_hbm.at[pl.ds(row0, CHUNK_ROWS)], src_v)
            pltpu.sync_copy(dst_hbm.at[pl.ds(row0, CHUNK_ROWS)], dst_v)
            pltpu.sync_copy(ew_hbm.at[pl.ds(row0, CHUNK_ROWS)], ew_v)
            for j in range(CHUNK_ROWS):
                for g in range(8):
                    sl = pl.ds(g * 16, 16)
                    xv = plsc.load_gather(x_v, [src_v[j, sl]])
                    msg_v[j, sl] = xv * ew_v[j, sl]
            descs = []
            for j in range(CHUNK_ROWS):
                descs.append(
                    pltpu.async_copy(msg_v.at[j], acc_sh.at[dst_v.at[j]],
                                     sem, add=True))
            for d in descs:
                d.wait()

        return carry

    lax.fori_loop(0, ITERS, chunk_body, 0)

    plsc.subcore_barrier()

    def dump(o_ref):
        def dchunk(off, size):
            pltpu.sync_copy(acc_sh.at[pl.ds(off, size)], z_v.at[pl.ds(0, size)])
            pltpu.sync_copy(z_v.at[pl.ds(0, size)], o_ref.at[pl.ds(off, size)])

        @pl.when(s < 15)
        def _():
            dchunk(s * 6400, 3200)
            dchunk(s * 6400 + 3200, 3200)

        @pl.when(s == 15)
        def _():
            dchunk(96000, 3200)
            dchunk(99200, 800)

    @pl.when(c == 0)
    def _():
        dump(o_y0)

    @pl.when(c == 1)
    def _():
        dump(o_y1)


@functools.partial(
    pl.kernel,
    out_type=jax.ShapeDtypeStruct((N * OUT_F,), _f32),
    mesh=_mesh,
    scratch_types=[
        pltpu.VMEM((PT,), _f32),
        pltpu.VMEM((PT,), _f32),
        pltpu.VMEM((PT,), _f32),
        pltpu.VMEM((PT,), _f32),
        pltpu.VMEM((PT,), _f32),
        pltpu.VMEM((PT,), _f32),
        pltpu.VMEM((KH, OUT_F), _f32),
        pltpu.VMEM((SUB * OUT_F,), _f32),
    ],
    compiler_params=_params,
)
def _proj(c0h, c1h, c2h, c3h, y0h, y1h, h_hbm, o_hbm,
          c0v, c1v, c2v, c3v, c4v, tmpv, hv, obuf):
    c = lax.axis_index("c")
    s = lax.axis_index("s")
    wid = s * 2 + c

    pltpu.sync_copy(h_hbm, hv)

    def load_body(seg, carry):
        off = wid * PT + seg * 800

        @pl.when(off < N)
        def _():
            lo = pl.ds(seg * 800, 800)
            ho = pl.ds(off, 800)
            pltpu.sync_copy(c0h.at[ho], c0v.at[lo])
            pltpu.sync_copy(c1h.at[ho], c1v.at[lo])
            pltpu.sync_copy(c2h.at[ho], c2v.at[lo])
            pltpu.sync_copy(c3h.at[ho], c3v.at[lo])
            pltpu.sync_copy(y0h.at[ho], c4v.at[lo])
            pltpu.sync_copy(y1h.at[ho], tmpv.at[lo])

        return carry

    lax.fori_loop(0, PT // 800, load_body, 0)

    def add_body(k, carry):
        sl = pl.ds(k * 16, 16)
        c4v[sl] = c4v[sl] + tmpv[sl]
        return carry

    lax.fori_loop(0, PT // 16, add_body, 0)

    hreg = [[hv[k, pl.ds(q * 16, 16)] for q in range(OUT_F // 16)]
            for k in range(KH)]

    def sub_body(sub, carry):
        node0 = wid * PT + sub * SUB

        @pl.when(node0 < N)
        def _():
            def node_body(n, inner):
                li = sub * SUB + n
                idxv = jnp.full((16,), li, jnp.int32)
                a0 = plsc.load_gather(c0v, [idxv])
                a1 = plsc.load_gather(c1v, [idxv])
                a2 = plsc.load_gather(c2v, [idxv])
                a3 = plsc.load_gather(c3v, [idxv])
                a4 = plsc.load_gather(c4v, [idxv])
                for q in range(OUT_F // 16):
                    acc = (a0 * hreg[0][q] + a1 * hreg[1][q]
                           + a2 * hreg[2][q] + a3 * hreg[3][q]
                           + a4 * hreg[4][q])
                    obuf[pl.ds(n * OUT_F + q * 16, 16)] = jnp.maximum(acc, 0.0)
                return inner

            lax.fori_loop(0, SUB, node_body, 0)
            pltpu.sync_copy(obuf, o_hbm.at[pl.ds(node0 * OUT_F, SUB * OUT_F)])

        return carry

    lax.fori_loop(0, PT // SUB, sub_body, 0)


def kernel(x, edge_index, edge_weights, weight):
    xf = x.reshape(N)
    src = edge_index[0].reshape(ROWS, 128)
    dst = edge_index[1].reshape(ROWS, 128)
    ew = edge_weights.reshape(ROWS, 128)
    h = weight.reshape(OUT_F, KH).T
    zer = jnp.zeros((N,), _f32)

    y0, y1 = xf, zer
    convs = []
    for _ in range(KH - 1):
        y0, y1, cv = _round(y0, y1, src, dst, ew)
        convs.append(cv)
    out = _proj(convs[0], convs[1], convs[2], convs[3], y0, y1, h)
    return out.reshape(1, N, OUT_F)
